# baseline (speedup 1.0000x reference)
import functools

import jax
import jax.numpy as jnp
from jax import lax
from jax.experimental import pallas as pl
from jax.experimental.pallas import tpu as pltpu
from jax.experimental.pallas import tpu_sc as plsc

NUM_SAMPLES = 10000

_KS1 = 42
_KS2 = 0x1BD11BDA ^ 42
_LANES = 128
_RPS = 8

_ROT_A = (13, 15, 26, 6)
_ROT_B = (17, 29, 16, 24)
_INJ = (
    (_KS1, (_KS2 + 1) & 0xFFFFFFFF),
    (_KS2, 2),
    (0, _KS1 + 3),
    (_KS1, (_KS2 + 4) & 0xFFFFFFFF),
    (_KS2, 5),
)


def _rotl(x, r):
    return (x << jnp.uint32(r)) | (x >> jnp.uint32(32 - r))


def _threefry_rounds(x0, x1):
    rots = (_ROT_A, _ROT_B, _ROT_A, _ROT_B, _ROT_A)
    for j in range(5):
        for r in rots[j]:
            x0 = x0 + x1
            x1 = _rotl(x1, r) ^ x0
        i0, i1 = _INJ[j]
        if i0:
            x0 = x0 + jnp.uint32(i0)
        x1 = x1 + jnp.uint32(i1)
    return x0, x1


def _sampler_body(hwbits, chunks, unc_ref, out_ref, r_ref):
    b = pl.program_id(0)
    nt = pl.program_id(1)
    nb = pl.num_programs(0)

    @pl.when(nt == 0)
    def _():
        r_ref[...] = 1.0 / unc_ref[0]

    sub = lax.broadcasted_iota(jnp.uint32, (_RPS, _LANES), 0)
    lane = lax.broadcasted_iota(jnp.uint32, (_RPS, _LANES), 1)
    m0 = (nt * _RPS * nb + b).astype(jnp.uint32)
    m_vec = m0 + sub * jnp.uint32(nb)
    hi = m_vec >> jnp.uint32(32 - hwbits)
    mask = jnp.uint32((1 << (32 - hwbits)) - 1)
    base = ((m_vec & mask) << jnp.uint32(hwbits)) + jnp.uint32(_KS1) + lane
    lane_i = lane.astype(jnp.int32)

    def body(ci, carry):
        best, bidx = carry
        coff = ci * _LANES
        x1 = base + coff.astype(jnp.uint32)
        x0 = hi + x1
        x1 = _rotl(x1, 13) ^ x0
        for r in (15, 26, 6):
            x0 = x0 + x1
            x1 = _rotl(x1, r) ^ x0
        x0 = x0 + jnp.uint32(_INJ[0][0])
        x1 = x1 + jnp.uint32(_INJ[0][1])
        rots = (_ROT_B, _ROT_A, _ROT_B, _ROT_A)
        for j in range(4):
            for r in rots[j]:
                x0 = x0 + x1
                x1 = _rotl(x1, r) ^ x0
            i0, i1 = _INJ[j + 1]
            if i0:
                x0 = x0 + jnp.uint32(i0)
            x1 = x1 + jnp.uint32(i1)
        bits = x0 ^ x1
        f = lax.bitcast_convert_type(
            (bits >> jnp.uint32(9)) | jnp.uint32(0x3F800000), jnp.float32)
        t = jnp.log2(f - 1.0)
        w = t * r_ref[pl.ds(ci, 1), :]
        cvec = lane_i + coff
        upd = w > best
        best = jnp.where(upd, w, best)
        bidx = jnp.where(upd, cvec, bidx)
        return best, bidx

    init = (jnp.full((_RPS, _LANES), -jnp.inf, jnp.float32),
            jnp.zeros((_RPS, _LANES), jnp.int32))
    best, bidx = lax.fori_loop(0, chunks, body, init)
    mx = jnp.max(best, axis=1, keepdims=True)
    cand = jnp.where(best == mx, bidx, jnp.int32(0x7FFFFFFF))
    out_ref[0, 0, :, :] = jnp.min(cand, axis=1, keepdims=True)


def _sample_indices(uncertainty, num_samples):
    b, h, w = uncertainty.shape
    hw = h * w
    hwbits = hw.bit_length() - 1
    assert (1 << hwbits) == hw and num_samples % _RPS == 0
    chunks = hw // _LANES
    nt = num_samples // _RPS
    unc3 = uncertainty.reshape(b, chunks, _LANES)
    out = pl.pallas_call(
        functools.partial(_sampler_body, hwbits, chunks),
        grid=(b, nt),
        in_specs=[pl.BlockSpec((1, chunks, _LANES), lambda bb, tt: (bb, 0, 0))],
        out_specs=pl.BlockSpec((1, 1, _RPS, 1), lambda bb, tt: (bb, tt, 0, 0)),
        out_shape=jax.ShapeDtypeStruct((b, nt, _RPS, 1), jnp.int32),
        scratch_shapes=[pltpu.VMEM((chunks, _LANES), jnp.float32)],
    )(unc3)
    return out.reshape(b, num_samples)


def _inv3x3_body(a_ref, o_ref):
    a = a_ref[...]
    c = [a[:, i:i + 1] for i in range(9)]
    a00, a01, a02, a10, a11, a12, a20, a21, a22 = c
    m00 = a11 * a22 - a12 * a21
    m01 = a12 * a20 - a10 * a22
    m02 = a10 * a21 - a11 * a20
    det = a00 * m00 + a01 * m01 + a02 * m02
    r = 1.0 / det
    cols = [m00 * r, (a02 * a21 - a01 * a22) * r, (a01 * a12 - a02 * a11) * r,
            m01 * r, (a00 * a22 - a02 * a20) * r, (a02 * a10 - a00 * a12) * r,
            m02 * r, (a01 * a20 - a00 * a21) * r, (a00 * a11 - a01 * a10) * r]
    cols += [jnp.zeros_like(det)] * 7
    o_ref[...] = jnp.concatenate(cols, axis=1)


def _invert_intrinsics(intrinsics):
    b = intrinsics.shape[0]
    return pl.pallas_call(
        _inv3x3_body,
        out_shape=jax.ShapeDtypeStruct((b, 16), jnp.float32),
    )(intrinsics.reshape(b, 9))


def _make_sc_gather(b, hw, w, pw):
    info = plsc.get_sparse_core_info()
    ncores = info.num_cores
    nwork = ncores * info.num_subcores
    assert nwork == 2 * b and pw % 16 == 0
    wbits = w.bit_length() - 1

    mesh = plsc.VectorSubcoreMesh(core_axis_name="c", subcore_axis_name="s")

    @functools.partial(
        pl.kernel, mesh=mesh,
        out_type=jax.ShapeDtypeStruct((nwork, 3, pw), jnp.float32),
        scratch_types=[
            pltpu.VMEM((pw,), jnp.int32),
            pltpu.VMEM((pw,), jnp.int32),
            pltpu.VMEM((pw,), jnp.float32),
            pltpu.VMEM((3, pw), jnp.float32),
            pltpu.VMEM((9, 16), jnp.float32),
            pltpu.SemaphoreType.DMA,
        ],
    )
    def sc_gather(dflat_hbm, kinv_hbm, idx_hbm, out_hbm,
                  idx_v, gidx_v, d_v, out_v, kv_v, sem):
        wid = lax.axis_index("s") * ncores + lax.axis_index("c")
        bb = wid // 2
        half = wid % 2
        pltpu.sync_copy(idx_hbm.at[pl.ds((bb * 2 + half) * pw, pw)], idx_v)
        pltpu.sync_copy(kinv_hbm.at[bb], kv_v)
        ioff = bb * hw

        def toglobal(k, _):
            pos = k * 16
            gidx_v[pl.ds(pos, 16)] = idx_v[pl.ds(pos, 16)] + ioff

        lax.fori_loop(0, pw // 16, toglobal, None, unroll=4)
        pltpu.async_copy(dflat_hbm.at[gidx_v], d_v, sem).wait()

        k00 = kv_v[0]
        k01 = kv_v[1]
        k02 = kv_v[2]
        k10 = kv_v[3]
        k11 = kv_v[4]
        k12 = kv_v[5]
        k20 = kv_v[6]
        k21 = kv_v[7]
        k22 = kv_v[8]
        def proj(k, _):
            pos = k * 16
            iv = idx_v[pl.ds(pos, 16)]
            d = d_v[pl.ds(pos, 16)]
            px = (iv & (w - 1)).astype(jnp.float32)
            py = (iv >> wbits).astype(jnp.float32)
            out_v[0, pl.ds(pos, 16)] = (k00 * px + k01 * py + k02) * d
            out_v[1, pl.ds(pos, 16)] = (k10 * px + k11 * py + k12) * d
            out_v[2, pl.ds(pos, 16)] = (k20 * px + k21 * py + k22) * d

        lax.fori_loop(0, pw // 16, proj, None, unroll=4)
        pltpu.sync_copy(out_v, out_hbm.at[wid])

    return sc_gather


def kernel(depth, intrinsics, uncertainty):
    b, h, w = depth.shape
    hw = h * w
    idx = _sample_indices(uncertainty, NUM_SAMPLES)
    kinv = _invert_intrinsics(intrinsics)
    kinv = jnp.repeat(kinv[:, :9, None], 16, axis=2)

    pw = (NUM_SAMPLES // 2 + 15) // 16 * 16
    pad = 2 * pw - NUM_SAMPLES
    idx_p = jnp.pad(idx, ((0, 0), (0, pad))).reshape(-1)
    dflat = depth.reshape(b * hw)
    sc = _make_sc_gather(b, hw, w, pw)
    pts = sc(dflat, kinv, idx_p).reshape(b, 2, 3, pw)
    pts = jnp.concatenate([pts[:, 0], pts[:, 1]], axis=-1)
    return jnp.swapaxes(pts[:, :, :NUM_SAMPLES], 1, 2)


# speedup vs baseline: 3.8508x; 3.8508x over previous
import functools

import jax
import jax.numpy as jnp
from jax import lax
from jax.experimental import pallas as pl
from jax.experimental.pallas import tpu as pltpu
from jax.experimental.pallas import tpu_sc as plsc

NUM_SAMPLES = 10000

_KS1 = 42
_KS2 = 0x1BD11BDA ^ 42
_LANES = 128
_RPS = 8
_UNROLL = 8

_ROT_A = (13, 15, 26, 6)
_ROT_B = (17, 29, 16, 24)
_INJ = (
    (_KS1, (_KS2 + 1) & 0xFFFFFFFF),
    (_KS2, 2),
    (0, _KS1 + 3),
    (_KS1, (_KS2 + 4) & 0xFFFFFFFF),
    (_KS2, 5),
)


def _rotl(x, r):
    return (x << jnp.uint32(r)) | (x >> jnp.uint32(32 - r))


def _threefry_rounds(x0, x1):
    rots = (_ROT_A, _ROT_B, _ROT_A, _ROT_B, _ROT_A)
    for j in range(5):
        for r in rots[j]:
            x0 = x0 + x1
            x1 = _rotl(x1, r) ^ x0
        i0, i1 = _INJ[j]
        if i0:
            x0 = x0 + jnp.uint32(i0)
        x1 = x1 + jnp.uint32(i1)
    return x0, x1


def _sampler_body(hwbits, chunks, unc_ref, out_ref, r_ref):
    b = pl.program_id(0)
    nt = pl.program_id(1)
    nb = pl.num_programs(0)

    @pl.when(nt == 0)
    def _():
        r_ref[...] = 1.0 / unc_ref[0]

    sub = lax.broadcasted_iota(jnp.uint32, (_RPS, _LANES), 0)
    lane = lax.broadcasted_iota(jnp.uint32, (_RPS, _LANES), 1)
    m0 = (nt * _RPS * nb + b).astype(jnp.uint32)
    m_vec = m0 + sub * jnp.uint32(nb)
    hi = m_vec >> jnp.uint32(32 - hwbits)
    mask = jnp.uint32((1 << (32 - hwbits)) - 1)
    base = ((m_vec & mask) << jnp.uint32(hwbits)) + jnp.uint32(_KS1) + lane
    lane_i = lane.astype(jnp.int32)

    def chunk_w(ci):
        coff = ci * _LANES
        x1 = base + coff.astype(jnp.uint32)
        x0 = hi + x1
        x1 = _rotl(x1, 13) ^ x0
        for r in (15, 26, 6):
            x0 = x0 + x1
            x1 = _rotl(x1, r) ^ x0
        x0 = x0 + jnp.uint32(_INJ[0][0])
        x1 = x1 + jnp.uint32(_INJ[0][1])
        rots = (_ROT_B, _ROT_A, _ROT_B, _ROT_A)
        for j in range(4):
            for r in rots[j]:
                x0 = x0 + x1
                x1 = _rotl(x1, r) ^ x0
            i0, i1 = _INJ[j + 1]
            if i0:
                x0 = x0 + jnp.uint32(i0)
            x1 = x1 + jnp.uint32(i1)
        bits = x0 ^ x1
        f = lax.bitcast_convert_type(
            (bits >> jnp.uint32(9)) | jnp.uint32(0x3F800000), jnp.float32)
        t = jnp.log2(f - 1.0)
        w = t * r_ref[pl.ds(ci, 1), :]
        return w, lane_i + coff

    def body(i, carry):
        out = []
        for k in range(_UNROLL):
            best, bidx = carry[k]
            w, cvec = chunk_w(i * _UNROLL + k)
            upd = w > best
            out.append((jnp.where(upd, w, best), jnp.where(upd, cvec, bidx)))
        return tuple(out)

    init = tuple((jnp.full((_RPS, _LANES), -jnp.inf, jnp.float32),
                  jnp.zeros((_RPS, _LANES), jnp.int32))
                 for _ in range(_UNROLL))
    accs = lax.fori_loop(0, chunks // _UNROLL, body, init)
    best, bidx = accs[0]
    for k in range(1, _UNROLL):
        wk, ck = accs[k]
        upd = (wk > best) | ((wk == best) & (ck < bidx))
        best = jnp.where(upd, wk, best)
        bidx = jnp.where(upd, ck, bidx)
    mx = jnp.max(best, axis=1, keepdims=True)
    cand = jnp.where(best == mx, bidx, jnp.int32(0x7FFFFFFF))
    out_ref[0, 0, :, :] = jnp.min(cand, axis=1, keepdims=True)


def _sample_indices(uncertainty, num_samples):
    b, h, w = uncertainty.shape
    hw = h * w
    hwbits = hw.bit_length() - 1
    assert (1 << hwbits) == hw and num_samples % _RPS == 0
    chunks = hw // _LANES
    nt = num_samples // _RPS
    unc3 = uncertainty.reshape(b, chunks, _LANES)
    out = pl.pallas_call(
        functools.partial(_sampler_body, hwbits, chunks),
        grid=(b, nt),
        in_specs=[pl.BlockSpec((1, chunks, _LANES), lambda bb, tt: (bb, 0, 0))],
        out_specs=pl.BlockSpec((1, 1, _RPS, 1), lambda bb, tt: (bb, tt, 0, 0)),
        out_shape=jax.ShapeDtypeStruct((b, nt, _RPS, 1), jnp.int32),
        scratch_shapes=[pltpu.VMEM((chunks, _LANES), jnp.float32)],
    )(unc3)
    return out.reshape(b, num_samples)


def _inv3x3_body(a_ref, o_ref):
    a = a_ref[...]
    c = [a[:, i:i + 1] for i in range(9)]
    a00, a01, a02, a10, a11, a12, a20, a21, a22 = c
    m00 = a11 * a22 - a12 * a21
    m01 = a12 * a20 - a10 * a22
    m02 = a10 * a21 - a11 * a20
    det = a00 * m00 + a01 * m01 + a02 * m02
    r = 1.0 / det
    cols = [m00 * r, (a02 * a21 - a01 * a22) * r, (a01 * a12 - a02 * a11) * r,
            m01 * r, (a00 * a22 - a02 * a20) * r, (a02 * a10 - a00 * a12) * r,
            m02 * r, (a01 * a20 - a00 * a21) * r, (a00 * a11 - a01 * a10) * r]
    cols += [jnp.zeros_like(det)] * 7
    o_ref[...] = jnp.concatenate(cols, axis=1)


def _invert_intrinsics(intrinsics):
    b = intrinsics.shape[0]
    return pl.pallas_call(
        _inv3x3_body,
        out_shape=jax.ShapeDtypeStruct((b, 16), jnp.float32),
    )(intrinsics.reshape(b, 9))


def _make_sc_gather(b, hw, w, pw):
    info = plsc.get_sparse_core_info()
    ncores = info.num_cores
    nwork = ncores * info.num_subcores
    assert nwork == 2 * b and pw % 16 == 0
    wbits = w.bit_length() - 1

    mesh = plsc.VectorSubcoreMesh(core_axis_name="c", subcore_axis_name="s")

    @functools.partial(
        pl.kernel, mesh=mesh,
        out_type=jax.ShapeDtypeStruct((nwork, 3, pw), jnp.float32),
        scratch_types=[
            pltpu.VMEM((pw,), jnp.int32),
            pltpu.VMEM((pw,), jnp.int32),
            pltpu.VMEM((pw,), jnp.float32),
            pltpu.VMEM((3, pw), jnp.float32),
            pltpu.VMEM((9, 16), jnp.float32),
            pltpu.SemaphoreType.DMA,
        ],
    )
    def sc_gather(dflat_hbm, kinv_hbm, idx_hbm, out_hbm,
                  idx_v, gidx_v, d_v, out_v, kv_v, sem):
        wid = lax.axis_index("s") * ncores + lax.axis_index("c")
        bb = wid // 2
        half = wid % 2
        pltpu.sync_copy(idx_hbm.at[pl.ds((bb * 2 + half) * pw, pw)], idx_v)
        pltpu.sync_copy(kinv_hbm.at[bb], kv_v)
        ioff = bb * hw

        def toglobal(k, _):
            pos = k * 16
            gidx_v[pl.ds(pos, 16)] = idx_v[pl.ds(pos, 16)] + ioff

        lax.fori_loop(0, pw // 16, toglobal, None, unroll=4)
        pltpu.async_copy(dflat_hbm.at[gidx_v], d_v, sem).wait()

        k00 = kv_v[0]
        k01 = kv_v[1]
        k02 = kv_v[2]
        k10 = kv_v[3]
        k11 = kv_v[4]
        k12 = kv_v[5]
        k20 = kv_v[6]
        k21 = kv_v[7]
        k22 = kv_v[8]
        def proj(k, _):
            pos = k * 16
            iv = idx_v[pl.ds(pos, 16)]
            d = d_v[pl.ds(pos, 16)]
            px = (iv & (w - 1)).astype(jnp.float32)
            py = (iv >> wbits).astype(jnp.float32)
            out_v[0, pl.ds(pos, 16)] = (k00 * px + k01 * py + k02) * d
            out_v[1, pl.ds(pos, 16)] = (k10 * px + k11 * py + k12) * d
            out_v[2, pl.ds(pos, 16)] = (k20 * px + k21 * py + k22) * d

        lax.fori_loop(0, pw // 16, proj, None, unroll=4)
        pltpu.sync_copy(out_v, out_hbm.at[wid])

    return sc_gather


def kernel(depth, intrinsics, uncertainty):
    b, h, w = depth.shape
    hw = h * w
    idx = _sample_indices(uncertainty, NUM_SAMPLES)
    kinv = _invert_intrinsics(intrinsics)
    kinv = jnp.repeat(kinv[:, :9, None], 16, axis=2)

    pw = (NUM_SAMPLES // 2 + 15) // 16 * 16
    pad = 2 * pw - NUM_SAMPLES
    idx_p = jnp.pad(idx, ((0, 0), (0, pad))).reshape(-1)
    dflat = depth.reshape(b * hw)
    sc = _make_sc_gather(b, hw, w, pw)
    pts = sc(dflat, kinv, idx_p).reshape(b, 2, 3, pw)
    pts = jnp.concatenate([pts[:, 0], pts[:, 1]], axis=-1)
    return jnp.swapaxes(pts[:, :, :NUM_SAMPLES], 1, 2)


# speedup vs baseline: 4.1099x; 1.0673x over previous
import functools

import jax
import jax.numpy as jnp
from jax import lax
from jax.experimental import pallas as pl
from jax.experimental.pallas import tpu as pltpu
from jax.experimental.pallas import tpu_sc as plsc

NUM_SAMPLES = 10000

_KS1 = 42
_KS2 = 0x1BD11BDA ^ 42
_LANES = 128
_RPS = 8
_UNROLL = 8

_ROT_A = (13, 15, 26, 6)
_ROT_B = (17, 29, 16, 24)
_INJ = (
    (_KS1, (_KS2 + 1) & 0xFFFFFFFF),
    (_KS2, 2),
    (0, _KS1 + 3),
    (_KS1, (_KS2 + 4) & 0xFFFFFFFF),
    (_KS2, 5),
)


def _rotl(x, r):
    return (x << jnp.uint32(r)) | (x >> jnp.uint32(32 - r))


def _threefry_rounds(x0, x1):
    rots = (_ROT_A, _ROT_B, _ROT_A, _ROT_B, _ROT_A)
    for j in range(5):
        for r in rots[j]:
            x0 = x0 + x1
            x1 = _rotl(x1, r) ^ x0
        i0, i1 = _INJ[j]
        if i0:
            x0 = x0 + jnp.uint32(i0)
        x1 = x1 + jnp.uint32(i1)
    return x0, x1


def _sampler_body(hwbits, chunks, unc_ref, out_ref, r_ref):
    b = pl.program_id(0)
    nt = pl.program_id(1)
    nb = pl.num_programs(0)

    @pl.when(nt == 0)
    def _():
        r_ref[...] = 1.0 / unc_ref[0]

    sub = lax.broadcasted_iota(jnp.uint32, (_RPS, _LANES), 0)
    lane = lax.broadcasted_iota(jnp.uint32, (_RPS, _LANES), 1)
    m0 = (nt * _RPS * nb + b).astype(jnp.uint32)
    m_vec = m0 + sub * jnp.uint32(nb)
    hi = m_vec >> jnp.uint32(32 - hwbits)
    mask = jnp.uint32((1 << (32 - hwbits)) - 1)
    base = ((m_vec & mask) << jnp.uint32(hwbits)) + jnp.uint32(_KS1) + lane
    lane_i = lane.astype(jnp.int32)

    def chunk_w(ci):
        coff = ci * _LANES
        x1 = base + coff.astype(jnp.uint32)
        x0 = hi + x1
        x1 = _rotl(x1, 13) ^ x0
        for r in (15, 26, 6):
            x0 = x0 + x1
            x1 = _rotl(x1, r) ^ x0
        x0 = x0 + jnp.uint32(_INJ[0][0])
        x1 = x1 + jnp.uint32(_INJ[0][1])
        rots = (_ROT_B, _ROT_A, _ROT_B, _ROT_A)
        for j in range(4):
            for r in rots[j]:
                x0 = x0 + x1
                x1 = _rotl(x1, r) ^ x0
            i0, i1 = _INJ[j + 1]
            if i0:
                x0 = x0 + jnp.uint32(i0)
            x1 = x1 + jnp.uint32(i1)
        bits = x0 ^ x1
        f = lax.bitcast_convert_type(
            (bits >> jnp.uint32(9)) | jnp.uint32(0x3F800000), jnp.float32)
        t = jnp.log2(f - 1.0)
        w = t * r_ref[pl.ds(ci, 1), :]
        return w, lane_i + coff

    def body(i, carry):
        pairs = [chunk_w(i * _UNROLL + k) for k in range(_UNROLL)]
        while len(pairs) > 1:
            nxt = []
            for a in range(0, len(pairs), 2):
                wa, ca = pairs[a]
                wb, cb = pairs[a + 1]
                upd = wb > wa
                nxt.append((jnp.where(upd, wb, wa), jnp.where(upd, cb, ca)))
            pairs = nxt
        best, bidx = carry
        w, cvec = pairs[0]
        upd = w > best
        return jnp.where(upd, w, best), jnp.where(upd, cvec, bidx)

    init = (jnp.full((_RPS, _LANES), -jnp.inf, jnp.float32),
            jnp.zeros((_RPS, _LANES), jnp.int32))
    best, bidx = lax.fori_loop(0, chunks // _UNROLL, body, init)
    mx = jnp.max(best, axis=1, keepdims=True)
    cand = jnp.where(best == mx, bidx, jnp.int32(0x7FFFFFFF))
    out_ref[0, 0, :, :] = jnp.min(cand, axis=1, keepdims=True)


def _sample_indices(uncertainty, num_samples):
    b, h, w = uncertainty.shape
    hw = h * w
    hwbits = hw.bit_length() - 1
    assert (1 << hwbits) == hw and num_samples % _RPS == 0
    chunks = hw // _LANES
    nt = num_samples // _RPS
    unc3 = uncertainty.reshape(b, chunks, _LANES)
    out = pl.pallas_call(
        functools.partial(_sampler_body, hwbits, chunks),
        grid=(b, nt),
        in_specs=[pl.BlockSpec((1, chunks, _LANES), lambda bb, tt: (bb, 0, 0))],
        out_specs=pl.BlockSpec((1, 1, _RPS, 1), lambda bb, tt: (bb, tt, 0, 0)),
        out_shape=jax.ShapeDtypeStruct((b, nt, _RPS, 1), jnp.int32),
        scratch_shapes=[pltpu.VMEM((chunks, _LANES), jnp.float32)],
    )(unc3)
    return out.reshape(b, num_samples)


def _inv3x3_body(a_ref, o_ref):
    a = a_ref[...]
    c = [a[:, i:i + 1] for i in range(9)]
    a00, a01, a02, a10, a11, a12, a20, a21, a22 = c
    m00 = a11 * a22 - a12 * a21
    m01 = a12 * a20 - a10 * a22
    m02 = a10 * a21 - a11 * a20
    det = a00 * m00 + a01 * m01 + a02 * m02
    r = 1.0 / det
    cols = [m00 * r, (a02 * a21 - a01 * a22) * r, (a01 * a12 - a02 * a11) * r,
            m01 * r, (a00 * a22 - a02 * a20) * r, (a02 * a10 - a00 * a12) * r,
            m02 * r, (a01 * a20 - a00 * a21) * r, (a00 * a11 - a01 * a10) * r]
    cols += [jnp.zeros_like(det)] * 7
    o_ref[...] = jnp.concatenate(cols, axis=1)


def _invert_intrinsics(intrinsics):
    b = intrinsics.shape[0]
    return pl.pallas_call(
        _inv3x3_body,
        out_shape=jax.ShapeDtypeStruct((b, 16), jnp.float32),
    )(intrinsics.reshape(b, 9))


def _make_sc_gather(b, hw, w, pw):
    info = plsc.get_sparse_core_info()
    ncores = info.num_cores
    nwork = ncores * info.num_subcores
    assert nwork == 2 * b and pw % 16 == 0
    wbits = w.bit_length() - 1

    mesh = plsc.VectorSubcoreMesh(core_axis_name="c", subcore_axis_name="s")

    @functools.partial(
        pl.kernel, mesh=mesh,
        out_type=jax.ShapeDtypeStruct((nwork, 3, pw), jnp.float32),
        scratch_types=[
            pltpu.VMEM((pw,), jnp.int32),
            pltpu.VMEM((pw,), jnp.int32),
            pltpu.VMEM((pw,), jnp.float32),
            pltpu.VMEM((3, pw), jnp.float32),
            pltpu.VMEM((9, 16), jnp.float32),
            pltpu.SemaphoreType.DMA,
        ],
    )
    def sc_gather(dflat_hbm, kinv_hbm, idx_hbm, out_hbm,
                  idx_v, gidx_v, d_v, out_v, kv_v, sem):
        wid = lax.axis_index("s") * ncores + lax.axis_index("c")
        bb = wid // 2
        half = wid % 2
        pltpu.sync_copy(idx_hbm.at[pl.ds((bb * 2 + half) * pw, pw)], idx_v)
        pltpu.sync_copy(kinv_hbm.at[bb], kv_v)
        ioff = bb * hw

        def toglobal(k, _):
            pos = k * 16
            gidx_v[pl.ds(pos, 16)] = idx_v[pl.ds(pos, 16)] + ioff

        lax.fori_loop(0, pw // 16, toglobal, None, unroll=4)
        pltpu.async_copy(dflat_hbm.at[gidx_v], d_v, sem).wait()

        k00 = kv_v[0]
        k01 = kv_v[1]
        k02 = kv_v[2]
        k10 = kv_v[3]
        k11 = kv_v[4]
        k12 = kv_v[5]
        k20 = kv_v[6]
        k21 = kv_v[7]
        k22 = kv_v[8]
        def proj(k, _):
            pos = k * 16
            iv = idx_v[pl.ds(pos, 16)]
            d = d_v[pl.ds(pos, 16)]
            px = (iv & (w - 1)).astype(jnp.float32)
            py = (iv >> wbits).astype(jnp.float32)
            out_v[0, pl.ds(pos, 16)] = (k00 * px + k01 * py + k02) * d
            out_v[1, pl.ds(pos, 16)] = (k10 * px + k11 * py + k12) * d
            out_v[2, pl.ds(pos, 16)] = (k20 * px + k21 * py + k22) * d

        lax.fori_loop(0, pw // 16, proj, None, unroll=4)
        pltpu.sync_copy(out_v, out_hbm.at[wid])

    return sc_gather


def kernel(depth, intrinsics, uncertainty):
    b, h, w = depth.shape
    hw = h * w
    idx = _sample_indices(uncertainty, NUM_SAMPLES)
    kinv = _invert_intrinsics(intrinsics)
    kinv = jnp.repeat(kinv[:, :9, None], 16, axis=2)

    pw = (NUM_SAMPLES // 2 + 15) // 16 * 16
    pad = 2 * pw - NUM_SAMPLES
    idx_p = jnp.pad(idx, ((0, 0), (0, pad))).reshape(-1)
    dflat = depth.reshape(b * hw)
    sc = _make_sc_gather(b, hw, w, pw)
    pts = sc(dflat, kinv, idx_p).reshape(b, 2, 3, pw)
    pts = jnp.concatenate([pts[:, 0], pts[:, 1]], axis=-1)
    return jnp.swapaxes(pts[:, :, :NUM_SAMPLES], 1, 2)
